# baseline (speedup 1.0000x reference)
import jax
import jax.numpy as jnp
from jax import lax
from jax.experimental import pallas as pl
from jax.experimental.pallas import tpu as pltpu
from jax.experimental.pallas import tpu_sc as plsc

N = 10000
E = 320000
NC, NS = 2, 16
NW = NC * NS
EW = E // NW
CH = 128
NCH = (EW + CH - 1) // CH
PAD = NCH * CH - EW
LAST = CH - PAD
N2 = N + 16
RPT = 624
TAIL = N - NS * RPT
NBUF0 = 6
NBUF1 = 8
BM = 2000
GRID = N // BM

def _mesh():
  return plsc.VectorSubcoreMesh(
      core_axis_name="c", subcore_axis_name="s", num_cores=NC, num_subcores=NS)


def _seg0(p0, src3, dst3, zer64, zer16, ones16):
  def body(tab, srcr, dstr, z64, z16, onesh, outa, outc,
           idxs, idxd, r0b, r1b, r2b, r3b, r4b, r5b, onesv, acc, cnt,
           s0m, s1m, s2m, s3m, s4m, s5m):
    c = lax.axis_index("c")
    s = lax.axis_index("s")
    wid = c * NS + s
    r0 = pl.multiple_of(s * RPT, 8)
    rows = [r0b, r1b, r2b, r3b, r4b, r5b]
    sems = [s0m, s1m, s2m, s3m, s4m, s5m]
    pltpu.sync_copy(srcr.at[wid], idxs)
    pltpu.sync_copy(dstr.at[wid], idxd)
    for b in range(NBUF0):
      pltpu.async_copy(tab.at[idxs.at[b]], rows[b], sems[b])
    pltpu.sync_copy(onesh, onesv)
    pltpu.sync_copy(z64.at[pl.ds(r0, RPT)], acc.at[pl.ds(r0, RPT)])
    pltpu.sync_copy(z16.at[pl.ds(r0, RPT)], cnt.at[pl.ds(r0, RPT)])

    @pl.when(s == NS - 1)
    def _():
      pltpu.sync_copy(z64.at[pl.ds(NS * RPT, TAIL)], acc.at[pl.ds(NS * RPT, TAIL)])
      pltpu.sync_copy(z16.at[pl.ds(NS * RPT, TAIL)], cnt.at[pl.ds(NS * RPT, TAIL)])

    plsc.subcore_barrier()

    def step(j, carry):
      base = j * NBUF0
      for b in range(NBUF0):
        i = base + b

        @pl.when(i < NCH)
        def _(i=i, b=b):
          pltpu.make_async_copy(tab.at[pl.ds(0, CH)], rows[b], sems[b]).wait()
          pltpu.sync_copy(rows[b], acc.at[idxd.at[i]], add=True)

          @pl.when(i < NCH - 1)
          def _():
            pltpu.sync_copy(onesv.at[pl.ds(0, CH)], cnt.at[idxd.at[i]], add=True)

          @pl.when(i == NCH - 1)
          def _():
            pltpu.sync_copy(onesv.at[pl.ds(CH, CH)], cnt.at[idxd.at[i]], add=True)

          @pl.when(i + NBUF0 < NCH)
          def _():
            pltpu.async_copy(tab.at[idxs.at[i + NBUF0]], rows[b], sems[b])

      return carry

    lax.fori_loop(0, (NCH + NBUF0 - 1) // NBUF0, step, 0)
    plsc.subcore_barrier()
    pltpu.sync_copy(acc.at[pl.ds(r0, RPT)], outa.at[c, pl.ds(r0, RPT)])
    pltpu.sync_copy(cnt.at[pl.ds(r0, RPT)], outc.at[c, pl.ds(r0, RPT)])

    @pl.when(s == NS - 1)
    def _():
      pltpu.sync_copy(acc.at[pl.ds(NS * RPT, TAIL)],
                      outa.at[c, pl.ds(NS * RPT, TAIL)])
      pltpu.sync_copy(cnt.at[pl.ds(NS * RPT, TAIL)],
                      outc.at[c, pl.ds(NS * RPT, TAIL)])

  f = pl.kernel(
      body,
      out_type=[jax.ShapeDtypeStruct((NC, N, 64), jnp.float32),
                jax.ShapeDtypeStruct((NC, N, 16), jnp.float32)],
      mesh=_mesh(),
      compiler_params=pltpu.CompilerParams(use_tc_tiling_on_sc=False),
      scratch_types=[
          pltpu.VMEM((NCH, CH), jnp.int32),
          pltpu.VMEM((NCH, CH), jnp.int32),
      ] + [pltpu.VMEM((CH, 64), jnp.float32)] * NBUF0 + [
          pltpu.VMEM((2 * CH, 16), jnp.float32),
          pltpu.VMEM_SHARED((N, 64), jnp.float32),
          pltpu.VMEM_SHARED((N, 16), jnp.float32),
      ] + [pltpu.SemaphoreType.DMA] * NBUF0,
  )
  return f(p0, src3, dst3, zer64, zer16, ones16)


def _seg1(h, src3, dst3, zer32):
  def body(tab, srcr, dstr, z32, outa, idxs, idxd,
           r0b, r1b, r2b, r3b, r4b, r5b, r6b, r7b, acc,
           s0m, s1m, s2m, s3m, s4m, s5m, s6m, s7m):
    c = lax.axis_index("c")
    s = lax.axis_index("s")
    wid = c * NS + s
    r0 = pl.multiple_of(s * RPT, 8)
    rows = [r0b, r1b, r2b, r3b, r4b, r5b, r6b, r7b]
    sems = [s0m, s1m, s2m, s3m, s4m, s5m, s6m, s7m]
    pltpu.sync_copy(srcr.at[wid], idxs)
    pltpu.sync_copy(dstr.at[wid], idxd)
    for b in range(NBUF1):
      pltpu.async_copy(tab.at[idxs.at[b]], rows[b], sems[b])
    pltpu.sync_copy(z32.at[pl.ds(r0, RPT)], acc.at[pl.ds(r0, RPT)])

    @pl.when(s == NS - 1)
    def _():
      pltpu.sync_copy(z32.at[pl.ds(NS * RPT, TAIL)], acc.at[pl.ds(NS * RPT, TAIL)])

    plsc.subcore_barrier()

    def step(j, carry):
      base = j * NBUF1
      for b in range(NBUF1):
        i = base + b

        @pl.when(i < NCH)
        def _(i=i, b=b):
          pltpu.make_async_copy(tab.at[pl.ds(0, CH)], rows[b], sems[b]).wait()
          pltpu.sync_copy(rows[b], acc.at[idxd.at[i]], add=True)

          @pl.when(i + NBUF1 < NCH)
          def _():
            pltpu.async_copy(tab.at[idxs.at[i + NBUF1]], rows[b], sems[b])

      return carry

    lax.fori_loop(0, (NCH + NBUF1 - 1) // NBUF1, step, 0)
    plsc.subcore_barrier()
    pltpu.sync_copy(acc.at[pl.ds(r0, RPT)], outa.at[c, pl.ds(r0, RPT)])

    @pl.when(s == NS - 1)
    def _():
      pltpu.sync_copy(acc.at[pl.ds(NS * RPT, TAIL)],
                      outa.at[c, pl.ds(NS * RPT, TAIL)])

  f = pl.kernel(
      body,
      out_type=jax.ShapeDtypeStruct((NC, N, 32), jnp.float32),
      mesh=_mesh(),
      compiler_params=pltpu.CompilerParams(use_tc_tiling_on_sc=False),
      scratch_types=[
          pltpu.VMEM((NCH, CH), jnp.int32),
          pltpu.VMEM((NCH, CH), jnp.int32),
      ] + [pltpu.VMEM((CH, 32), jnp.float32)] * NBUF1 + [
          pltpu.VMEM_SHARED((N, 32), jnp.float32),
      ] + [pltpu.SemaphoreType.DMA] * NBUF1,
  )
  return f(h, src3, dst3, zer32)


def _tc0a(x, Wa):
  def body(x_ref, wa_ref, p_ref):
    p_ref[...] = jnp.dot(x_ref[...], wa_ref[...],
                         preferred_element_type=jnp.float32)

  return pl.pallas_call(
      body,
      grid=(GRID,),
      in_specs=[
          pl.BlockSpec((BM, 128), lambda i: (i, 0)),
          pl.BlockSpec((128, 64), lambda i: (0, 0)),
      ],
      out_specs=pl.BlockSpec((BM, 64), lambda i: (i, 0)),
      out_shape=jax.ShapeDtypeStruct((N, 64), jnp.float32),
  )(x, Wa)


def _tc0b(x, Wb, bb):
  def body(x_ref, wb_ref, bb_ref, sb_ref):
    sb_ref[...] = jnp.dot(x_ref[...], wb_ref[...],
                          preferred_element_type=jnp.float32) + bb_ref[...]

  return pl.pallas_call(
      body,
      grid=(GRID,),
      in_specs=[
          pl.BlockSpec((BM, 128), lambda i: (i, 0)),
          pl.BlockSpec((128, 64), lambda i: (0, 0)),
          pl.BlockSpec((1, 64), lambda i: (0, 0)),
      ],
      out_specs=pl.BlockSpec((BM, 64), lambda i: (i, 0)),
      out_shape=jax.ShapeDtypeStruct((N, 64), jnp.float32),
  )(x, Wb, bb)


def _tc1(s0a, s0c, p0, sb0, bfr, bbe, wc1, bc1):
  def body(sa_ref, sc_ref, p_ref, sb_ref, bfr_ref, bbe_ref, wc_ref, bc_ref,
           h_ref, aux_ref):
    sa = sa_ref[...]
    ssum = sa[0] + sa[1] + p_ref[...]
    scv = sc_ref[...]
    cnt = scv[0, :, 0:1] + scv[1, :, 0:1] + 1.0
    mean = ssum / cnt
    fr = mean[:, :32] + bfr_ref[...]
    be = mean[:, 32:] + bbe_ref[...]
    sb = sb_ref[...]
    bal = jax.nn.sigmoid(sb[:, 32:33])
    g = jnp.maximum(fr * bal + be * (1.0 - bal), 0.0)
    h = jnp.maximum(sb[:, :32] + g, 0.0)
    h_ref[...] = h
    hy = jnp.dot(h, wc_ref[...], preferred_element_type=jnp.float32) + bc_ref[...]
    col = lax.broadcasted_iota(jnp.int32, hy.shape, 1)
    aux_ref[...] = jnp.where(col < 64, hy,
                             jnp.where(col == 64, jax.nn.sigmoid(hy), cnt))

  return pl.pallas_call(
      body,
      grid=(GRID,),
      in_specs=[
          pl.BlockSpec((NC, BM, 64), lambda i: (0, i, 0)),
          pl.BlockSpec((NC, BM, 16), lambda i: (0, i, 0)),
          pl.BlockSpec((BM, 64), lambda i: (i, 0)),
          pl.BlockSpec((BM, 64), lambda i: (i, 0)),
          pl.BlockSpec((1, 32), lambda i: (0, 0)),
          pl.BlockSpec((1, 32), lambda i: (0, 0)),
          pl.BlockSpec((32, 128), lambda i: (0, 0)),
          pl.BlockSpec((1, 128), lambda i: (0, 0)),
      ],
      out_specs=[pl.BlockSpec((BM, 32), lambda i: (i, 0)),
                 pl.BlockSpec((BM, 128), lambda i: (i, 0))],
      out_shape=[jax.ShapeDtypeStruct((N, 32), jnp.float32),
                 jax.ShapeDtypeStruct((N, 128), jnp.float32)],
  )(s0a, s0c, p0, sb0, bfr, bbe, wc1, bc1)


def _tc2(s1a, h32, aux, wd1, bd1):
  def body(sa_ref, h_ref, aux_ref, wd_ref, bd_ref, out_ref):
    sa = sa_ref[...]
    s1 = sa[0] + sa[1] + h_ref[...]
    auxv = aux_ref[...]
    cnt = auxv[:, 65:66]
    mean1 = s1 / cnt
    mm = jnp.dot(mean1, wd_ref[...],
                 preferred_element_type=jnp.float32) + bd_ref[...]
    fr1 = mm[:, :64]
    be1 = mm[:, 64:]
    bal1 = auxv[:, 64:65]
    out_ref[...] = auxv[:, :64] + jnp.maximum(
        fr1 * bal1 + be1 * (1.0 - bal1), 0.0)

  return pl.pallas_call(
      body,
      grid=(GRID,),
      in_specs=[
          pl.BlockSpec((NC, BM, 32), lambda i: (0, i, 0)),
          pl.BlockSpec((BM, 32), lambda i: (i, 0)),
          pl.BlockSpec((BM, 128), lambda i: (i, 0)),
          pl.BlockSpec((32, 128), lambda i: (0, 0)),
          pl.BlockSpec((1, 128), lambda i: (0, 0)),
      ],
      out_specs=pl.BlockSpec((BM, 64), lambda i: (i, 0)),
      out_shape=jax.ShapeDtypeStruct((N, 64), jnp.float32),
  )(s1a, h32, aux, wd1, bd1)


def kernel(x, edge_index, W1, b1, Wbe0, bbe0, Wfr0, bfr0, Wbal0, bbal0,
           Wself0, bself0, Wbe1, bbe1, Wfr1, bfr1, Wbal1, bbal1,
           Wself1, bself1):
  f32 = jnp.float32
  i32 = jnp.int32
  src_pad = jnp.broadcast_to(
      N + (jnp.arange(NW, dtype=i32) % NS)[:, None], (NW, PAD))
  dst_pad = jnp.zeros((NW, PAD), i32)
  src3 = jnp.concatenate([edge_index[0].reshape(NW, EW), src_pad],
                         axis=1).reshape(NW, NCH, CH)
  dst3 = jnp.concatenate([edge_index[1].reshape(NW, EW), dst_pad],
                         axis=1).reshape(NW, NCH, CH)

  Wa0 = jnp.concatenate([Wfr0, Wbe0], axis=1)
  Wb0 = jnp.concatenate([Wself0, Wbal0, jnp.zeros((128, 31), f32)], axis=1)
  bb0 = jnp.concatenate([bself0, bbal0, jnp.zeros((31,), f32)]).reshape(1, 64)
  p0 = _tc0a(x, Wa0)
  p0z = jnp.concatenate([p0, jnp.zeros((N2 - N, 64), f32)])

  zer64 = jnp.zeros((N, 64), f32)
  zer32 = jnp.zeros((N, 32), f32)
  zer16 = jnp.zeros((N, 16), f32)
  ones16 = jnp.concatenate(
      [jnp.ones((CH + LAST, 16), f32), jnp.zeros((PAD, 16), f32)])
  s0a, s0c = _seg0(p0z, src3, dst3, zer64, zer16, ones16)
  sb0 = _tc0b(x, Wb0, bb0)

  wc1 = jnp.concatenate([Wself1, Wbal1, jnp.zeros((32, 63), f32)], axis=1)
  bc1 = jnp.concatenate([bself1, bbal1, jnp.zeros((63,), f32)]).reshape(1, 128)
  h32, aux = _tc1(s0a, s0c, p0, sb0, bfr0.reshape(1, 32), bbe0.reshape(1, 32),
                  wc1, bc1)

  h32z = jnp.concatenate([h32, jnp.zeros((N2 - N, 32), f32)])
  s1a = _seg1(h32z, src3, dst3, zer32)

  wd1 = jnp.concatenate([Wfr1, Wbe1], axis=1)
  bd1 = jnp.concatenate([bfr1, bbe1]).reshape(1, 128)
  out = _tc2(s1a, h32, aux, wd1, bd1)
  return (h32, out)


# speedup vs baseline: 1.0765x; 1.0765x over previous
import jax
import jax.numpy as jnp
from jax import lax
from jax.experimental import pallas as pl
from jax.experimental.pallas import tpu as pltpu
from jax.experimental.pallas import tpu_sc as plsc

N = 10000
E = 320000
NC, NS = 2, 16
NW = NC * NS
EW = E // NW
CH = 80
NCH = EW // CH
RPT = 624
TAIL = N - NS * RPT
NBUF = 8
BM = 2000
GRID = N // BM

def _mesh():
  return plsc.VectorSubcoreMesh(
      core_axis_name="c", subcore_axis_name="s", num_cores=NC, num_subcores=NS)


def _seg0(p0, src3, dst3, zer64, zer16, ones16):
  def body(tab, srcr, dstr, z64, z16, onesh, outa, outc,
           idxs, idxd, r0b, r1b, r2b, r3b, r4b, r5b, r6b, r7b, onesv, acc, cnt,
           s0m, s1m, s2m, s3m, s4m, s5m, s6m, s7m):
    c = lax.axis_index("c")
    s = lax.axis_index("s")
    wid = c * NS + s
    r0 = pl.multiple_of(s * RPT, 8)
    rows = [r0b, r1b, r2b, r3b, r4b, r5b, r6b, r7b]
    sems = [s0m, s1m, s2m, s3m, s4m, s5m, s6m, s7m]
    pltpu.sync_copy(srcr.at[wid], idxs)
    pltpu.sync_copy(dstr.at[wid], idxd)
    for b in range(NBUF):
      pltpu.async_copy(tab.at[idxs.at[b]], rows[b], sems[b])
    pltpu.sync_copy(onesh, onesv)
    pltpu.sync_copy(z64.at[pl.ds(r0, RPT)], acc.at[pl.ds(r0, RPT)])
    pltpu.sync_copy(z16.at[pl.ds(r0, RPT)], cnt.at[pl.ds(r0, RPT)])

    @pl.when(s == NS - 1)
    def _():
      pltpu.sync_copy(z64.at[pl.ds(NS * RPT, TAIL)], acc.at[pl.ds(NS * RPT, TAIL)])
      pltpu.sync_copy(z16.at[pl.ds(NS * RPT, TAIL)], cnt.at[pl.ds(NS * RPT, TAIL)])

    plsc.subcore_barrier()

    def step(j, carry):
      base = j * NBUF
      for b in range(NBUF):
        i = base + b

        @pl.when(i < NCH)
        def _(i=i, b=b):
          pltpu.make_async_copy(tab.at[pl.ds(0, CH)], rows[b], sems[b]).wait()
          pltpu.sync_copy(rows[b], acc.at[idxd.at[i]], add=True)
          pltpu.sync_copy(onesv, cnt.at[idxd.at[i]], add=True)

          @pl.when(i + NBUF < NCH)
          def _():
            pltpu.async_copy(tab.at[idxs.at[i + NBUF]], rows[b], sems[b])

      return carry

    lax.fori_loop(0, (NCH + NBUF - 1) // NBUF, step, 0)
    plsc.subcore_barrier()
    pltpu.sync_copy(acc.at[pl.ds(r0, RPT)], outa.at[c, pl.ds(r0, RPT)])
    pltpu.sync_copy(cnt.at[pl.ds(r0, RPT)], outc.at[c, pl.ds(r0, RPT)])

    @pl.when(s == NS - 1)
    def _():
      pltpu.sync_copy(acc.at[pl.ds(NS * RPT, TAIL)],
                      outa.at[c, pl.ds(NS * RPT, TAIL)])
      pltpu.sync_copy(cnt.at[pl.ds(NS * RPT, TAIL)],
                      outc.at[c, pl.ds(NS * RPT, TAIL)])

  f = pl.kernel(
      body,
      out_type=[jax.ShapeDtypeStruct((NC, N, 64), jnp.float32),
                jax.ShapeDtypeStruct((NC, N, 16), jnp.float32)],
      mesh=_mesh(),
      compiler_params=pltpu.CompilerParams(use_tc_tiling_on_sc=False),
      scratch_types=[
          pltpu.VMEM((NCH, CH), jnp.int32),
          pltpu.VMEM((NCH, CH), jnp.int32),
      ] + [pltpu.VMEM((CH, 64), jnp.float32)] * NBUF + [
          pltpu.VMEM((CH, 16), jnp.float32),
          pltpu.VMEM_SHARED((N, 64), jnp.float32),
          pltpu.VMEM_SHARED((N, 16), jnp.float32),
      ] + [pltpu.SemaphoreType.DMA] * NBUF,
  )
  return f(p0, src3, dst3, zer64, zer16, ones16)


def _seg1(h, src3, dst3, zer32):
  def body(tab, srcr, dstr, z32, outa, idxs, idxd,
           r0b, r1b, r2b, r3b, r4b, r5b, r6b, r7b, acc,
           s0m, s1m, s2m, s3m, s4m, s5m, s6m, s7m):
    c = lax.axis_index("c")
    s = lax.axis_index("s")
    wid = c * NS + s
    r0 = pl.multiple_of(s * RPT, 8)
    rows = [r0b, r1b, r2b, r3b, r4b, r5b, r6b, r7b]
    sems = [s0m, s1m, s2m, s3m, s4m, s5m, s6m, s7m]
    pltpu.sync_copy(srcr.at[wid], idxs)
    pltpu.sync_copy(dstr.at[wid], idxd)
    for b in range(NBUF):
      pltpu.async_copy(tab.at[idxs.at[b]], rows[b], sems[b])
    pltpu.sync_copy(z32.at[pl.ds(r0, RPT)], acc.at[pl.ds(r0, RPT)])

    @pl.when(s == NS - 1)
    def _():
      pltpu.sync_copy(z32.at[pl.ds(NS * RPT, TAIL)], acc.at[pl.ds(NS * RPT, TAIL)])

    plsc.subcore_barrier()

    def step(j, carry):
      base = j * NBUF
      for b in range(NBUF):
        i = base + b

        @pl.when(i < NCH)
        def _(i=i, b=b):
          pltpu.make_async_copy(tab.at[pl.ds(0, CH)], rows[b], sems[b]).wait()
          pltpu.sync_copy(rows[b], acc.at[idxd.at[i]], add=True)

          @pl.when(i + NBUF < NCH)
          def _():
            pltpu.async_copy(tab.at[idxs.at[i + NBUF]], rows[b], sems[b])

      return carry

    lax.fori_loop(0, (NCH + NBUF - 1) // NBUF, step, 0)
    plsc.subcore_barrier()
    pltpu.sync_copy(acc.at[pl.ds(r0, RPT)], outa.at[c, pl.ds(r0, RPT)])

    @pl.when(s == NS - 1)
    def _():
      pltpu.sync_copy(acc.at[pl.ds(NS * RPT, TAIL)],
                      outa.at[c, pl.ds(NS * RPT, TAIL)])

  f = pl.kernel(
      body,
      out_type=jax.ShapeDtypeStruct((NC, N, 32), jnp.float32),
      mesh=_mesh(),
      compiler_params=pltpu.CompilerParams(use_tc_tiling_on_sc=False),
      scratch_types=[
          pltpu.VMEM((NCH, CH), jnp.int32),
          pltpu.VMEM((NCH, CH), jnp.int32),
      ] + [pltpu.VMEM((CH, 32), jnp.float32)] * NBUF + [
          pltpu.VMEM_SHARED((N, 32), jnp.float32),
      ] + [pltpu.SemaphoreType.DMA] * NBUF,
  )
  return f(h, src3, dst3, zer32)


def _tc0a(x, Wa):
  def body(x_ref, wa_ref, p_ref):
    p_ref[...] = jnp.dot(x_ref[...], wa_ref[...],
                         preferred_element_type=jnp.float32)

  return pl.pallas_call(
      body,
      grid=(GRID,),
      in_specs=[
          pl.BlockSpec((BM, 128), lambda i: (i, 0)),
          pl.BlockSpec((128, 64), lambda i: (0, 0)),
      ],
      out_specs=pl.BlockSpec((BM, 64), lambda i: (i, 0)),
      out_shape=jax.ShapeDtypeStruct((N, 64), jnp.float32),
  )(x, Wa)


def _tc0b(x, Wb, bb):
  def body(x_ref, wb_ref, bb_ref, sb_ref):
    sb_ref[...] = jnp.dot(x_ref[...], wb_ref[...],
                          preferred_element_type=jnp.float32) + bb_ref[...]

  return pl.pallas_call(
      body,
      grid=(GRID,),
      in_specs=[
          pl.BlockSpec((BM, 128), lambda i: (i, 0)),
          pl.BlockSpec((128, 64), lambda i: (0, 0)),
          pl.BlockSpec((1, 64), lambda i: (0, 0)),
      ],
      out_specs=pl.BlockSpec((BM, 64), lambda i: (i, 0)),
      out_shape=jax.ShapeDtypeStruct((N, 64), jnp.float32),
  )(x, Wb, bb)


def _tc1(s0a, s0c, p0, sb0, bfr, bbe):
  def body(sa_ref, sc_ref, p_ref, sb_ref, bfr_ref, bbe_ref, h_ref):
    sa = sa_ref[...]
    ssum = sa[0] + sa[1] + p_ref[...]
    scv = sc_ref[...]
    cnt = scv[0, :, 0:1] + scv[1, :, 0:1] + 1.0
    mean = ssum / cnt
    fr = mean[:, :32] + bfr_ref[...]
    be = mean[:, 32:] + bbe_ref[...]
    sb = sb_ref[...]
    bal = jax.nn.sigmoid(sb[:, 32:33])
    g = jnp.maximum(fr * bal + be * (1.0 - bal), 0.0)
    h_ref[...] = jnp.maximum(sb[:, :32] + g, 0.0)

  return pl.pallas_call(
      body,
      grid=(GRID,),
      in_specs=[
          pl.BlockSpec((NC, BM, 64), lambda i: (0, i, 0)),
          pl.BlockSpec((NC, BM, 16), lambda i: (0, i, 0)),
          pl.BlockSpec((BM, 64), lambda i: (i, 0)),
          pl.BlockSpec((BM, 64), lambda i: (i, 0)),
          pl.BlockSpec((1, 32), lambda i: (0, 0)),
          pl.BlockSpec((1, 32), lambda i: (0, 0)),
      ],
      out_specs=pl.BlockSpec((BM, 32), lambda i: (i, 0)),
      out_shape=jax.ShapeDtypeStruct((N, 32), jnp.float32),
  )(s0a, s0c, p0, sb0, bfr, bbe)


def _tc2(s1a, s0c, h32, wc1, bc1, wd1, bd1):
  def body(sa_ref, sc_ref, h_ref, wc_ref, bc_ref, wd_ref, bd_ref, out_ref):
    h = h_ref[...]
    scv = sc_ref[...]
    cnt = scv[0, :, 0:1] + scv[1, :, 0:1] + 1.0
    hy = jnp.dot(h, wc_ref[...], preferred_element_type=jnp.float32) + bc_ref[...]
    bal1 = jax.nn.sigmoid(hy[:, 64:65])
    sa = sa_ref[...]
    s1 = sa[0] + sa[1] + h
    mean1 = s1 / cnt
    mm = jnp.dot(mean1, wd_ref[...],
                 preferred_element_type=jnp.float32) + bd_ref[...]
    fr1 = mm[:, :64]
    be1 = mm[:, 64:]
    out_ref[...] = hy[:, :64] + jnp.maximum(
        fr1 * bal1 + be1 * (1.0 - bal1), 0.0)

  return pl.pallas_call(
      body,
      grid=(GRID,),
      in_specs=[
          pl.BlockSpec((NC, BM, 32), lambda i: (0, i, 0)),
          pl.BlockSpec((NC, BM, 16), lambda i: (0, i, 0)),
          pl.BlockSpec((BM, 32), lambda i: (i, 0)),
          pl.BlockSpec((32, 128), lambda i: (0, 0)),
          pl.BlockSpec((1, 128), lambda i: (0, 0)),
          pl.BlockSpec((32, 128), lambda i: (0, 0)),
          pl.BlockSpec((1, 128), lambda i: (0, 0)),
      ],
      out_specs=pl.BlockSpec((BM, 64), lambda i: (i, 0)),
      out_shape=jax.ShapeDtypeStruct((N, 64), jnp.float32),
  )(s1a, s0c, h32, wc1, bc1, wd1, bd1)


def kernel(x, edge_index, W1, b1, Wbe0, bbe0, Wfr0, bfr0, Wbal0, bbal0,
           Wself0, bself0, Wbe1, bbe1, Wfr1, bfr1, Wbal1, bbal1,
           Wself1, bself1):
  f32 = jnp.float32
  src3 = edge_index[0].reshape(NW, NCH, CH)
  dst3 = edge_index[1].reshape(NW, NCH, CH)

  Wa0 = jnp.concatenate([Wfr0, Wbe0], axis=1)
  Wb0 = jnp.concatenate([Wself0, Wbal0, jnp.zeros((128, 31), f32)], axis=1)
  bb0 = jnp.concatenate([bself0, bbal0, jnp.zeros((31,), f32)]).reshape(1, 64)
  p0 = _tc0a(x, Wa0)

  zer64 = jnp.zeros((N, 64), f32)
  zer32 = jnp.zeros((N, 32), f32)
  zer16 = jnp.zeros((N, 16), f32)
  ones16 = jnp.ones((CH, 16), f32)
  s0a, s0c = _seg0(p0, src3, dst3, zer64, zer16, ones16)
  sb0 = _tc0b(x, Wb0, bb0)

  h32 = _tc1(s0a, s0c, p0, sb0, bfr0.reshape(1, 32), bbe0.reshape(1, 32))

  s1a = _seg1(h32, src3, dst3, zer32)

  wc1 = jnp.concatenate([Wself1, Wbal1, jnp.zeros((32, 63), f32)], axis=1)
  bc1 = jnp.concatenate([bself1, bbal1, jnp.zeros((63,), f32)]).reshape(1, 128)
  wd1 = jnp.concatenate([Wfr1, Wbe1], axis=1)
  bd1 = jnp.concatenate([bfr1, bbe1]).reshape(1, 128)
  out = _tc2(s1a, s0c, h32, wc1, bc1, wd1, bd1)
  return (h32, out)
